# baseline (speedup 1.0000x reference)
import jax
import jax.numpy as jnp
from jax import lax
from jax.experimental import pallas as pl
from jax.experimental.pallas import tpu as pltpu

B, S, D = 4096, 200, 128

TC_BLK = 64
TC_NB = 6
TC_STEPS = B // TC_BLK


def _tc_body(x_hbm, w_hbm, o_ref, xbuf, wbuf, xsems, wsems):
    i = pl.program_id(0)

    def fetch(blk, slot):
        rows = pl.ds(blk * TC_BLK, TC_BLK)
        pltpu.make_async_copy(x_hbm.at[rows], xbuf.at[slot], xsems.at[slot]).start()
        pltpu.make_async_copy(w_hbm.at[rows], wbuf.at[slot], wsems.at[slot]).start()

    @pl.when(i == 0)
    def _():
        for b in range(TC_NB):
            fetch(b, b)

    slot = lax.rem(i, TC_NB)
    nxt = i + TC_NB

    pltpu.make_async_copy(x_hbm.at[pl.ds(0, TC_BLK)], xbuf.at[slot], xsems.at[slot]).wait()
    pltpu.make_async_copy(w_hbm.at[pl.ds(0, TC_BLK)], wbuf.at[slot], wsems.at[slot]).wait()

    x = xbuf[slot]
    w = wbuf[slot]
    num = jnp.sum(x * w[:, :, None], axis=1)
    den = jnp.sum(w, axis=1)
    o_ref[...] = num / den[:, None]

    @pl.when(nxt < TC_STEPS)
    def _():
        fetch(nxt, slot)


@jax.jit
def _run(x, w):
    return pl.pallas_call(
        _tc_body,
        grid=(TC_STEPS,),
        in_specs=[
            pl.BlockSpec(memory_space=pl.ANY),
            pl.BlockSpec(memory_space=pl.ANY),
        ],
        out_specs=pl.BlockSpec((TC_BLK, D), lambda i: (i, 0)),
        out_shape=jax.ShapeDtypeStruct((B, D), jnp.float32),
        scratch_shapes=[
            pltpu.VMEM((TC_NB, TC_BLK, S, D), jnp.float32),
            pltpu.VMEM((TC_NB, TC_BLK, S), jnp.float32),
            pltpu.SemaphoreType.DMA((TC_NB,)),
            pltpu.SemaphoreType.DMA((TC_NB,)),
        ],
    )(x, w)


def kernel(inputs, item_id_seq_weight):
    return _run(inputs, item_id_seq_weight.astype(jnp.float32))


# speedup vs baseline: 1.0123x; 1.0123x over previous
import jax
import jax.numpy as jnp
from jax import lax
from jax.experimental import pallas as pl
from jax.experimental.pallas import tpu as pltpu

B, S, D = 4096, 200, 128

TC_BLK = 128
TC_NB = 3
TC_STEPS = B // TC_BLK


def _tc_body(x_hbm, w_hbm, o_ref, xbuf, wbuf, xsems, wsems):
    i = pl.program_id(0)

    def fetch(blk, slot):
        rows = pl.ds(blk * TC_BLK, TC_BLK)
        pltpu.make_async_copy(x_hbm.at[rows], xbuf.at[slot], xsems.at[slot]).start()
        pltpu.make_async_copy(w_hbm.at[rows], wbuf.at[slot], wsems.at[slot]).start()

    @pl.when(i == 0)
    def _():
        for b in range(TC_NB):
            fetch(b, b)

    slot = lax.rem(i, TC_NB)
    nxt = i + TC_NB

    pltpu.make_async_copy(x_hbm.at[pl.ds(0, TC_BLK)], xbuf.at[slot], xsems.at[slot]).wait()
    pltpu.make_async_copy(w_hbm.at[pl.ds(0, TC_BLK)], wbuf.at[slot], wsems.at[slot]).wait()

    x = xbuf[slot]
    w = wbuf[slot]
    num = jnp.sum(x * w[:, :, None], axis=1)
    den = jnp.sum(w, axis=1)
    o_ref[...] = num / den[:, None]

    @pl.when(nxt < TC_STEPS)
    def _():
        fetch(nxt, slot)


@jax.jit
def _run(x, w):
    return pl.pallas_call(
        _tc_body,
        grid=(TC_STEPS,),
        in_specs=[
            pl.BlockSpec(memory_space=pl.ANY),
            pl.BlockSpec(memory_space=pl.ANY),
        ],
        out_specs=pl.BlockSpec((TC_BLK, D), lambda i: (i, 0)),
        out_shape=jax.ShapeDtypeStruct((B, D), jnp.float32),
        scratch_shapes=[
            pltpu.VMEM((TC_NB, TC_BLK, S, D), jnp.float32),
            pltpu.VMEM((TC_NB, TC_BLK, S), jnp.float32),
            pltpu.SemaphoreType.DMA((TC_NB,)),
            pltpu.SemaphoreType.DMA((TC_NB,)),
        ],
    )(x, w)


def kernel(inputs, item_id_seq_weight):
    return _run(inputs, item_id_seq_weight.astype(jnp.float32))
